# baseline (speedup 1.0000x reference)
import functools

import jax
import jax.numpy as jnp
from jax import lax
from jax.experimental import pallas as pl
from jax.experimental.pallas import tpu as pltpu
from jax.experimental.pallas import tpu_sc as plsc

ACT = 2048
DICT = 16384
TOKENS = 4096
TOPK_PER_TOKEN = 64
TOTAL = TOKENS * DICT
TOTAL_K = TOKENS * TOPK_PER_TOKEN


_EBM, _EBN = 1024, 1024


def _encode_body(x_ref, w_ref, b_ref, o_ref):
    pre = (jnp.dot(x_ref[...], w_ref[...], preferred_element_type=jnp.float32)
           + b_ref[...])
    o_ref[...] = jnp.where(pre > 0, pre, 0.0)


def _encode(x, W_enc, b_enc):
    grid = (TOKENS // _EBM, DICT // _EBN)
    return pl.pallas_call(
        _encode_body,
        grid=grid,
        in_specs=[
            pl.BlockSpec((_EBM, ACT), lambda i, j: (i, 0)),
            pl.BlockSpec((ACT, _EBN), lambda i, j: (0, j)),
            pl.BlockSpec((1, _EBN), lambda i, j: (0, j)),
        ],
        out_specs=pl.BlockSpec((_EBM, _EBN), lambda i, j: (i, j)),
        out_shape=jax.ShapeDtypeStruct((TOKENS, DICT), jnp.float32),
    )(x, W_enc, b_enc.reshape(1, DICT))


NB = 4096
LANES = 16
HWORDS = NB * LANES
NW = 32
CHUNK = DICT
ROWS_W = TOKENS // NW
NCHUNK = ROWS_W
_UNROLL = 16


def _hist_body(shift, masked, acts_hbm, lo_hbm, hi_hbm, hist_hbm,
               lo_v, hi_v, buf0, buf1, hist_v, hred_v, sem0, sem1):
    wid = lax.axis_index("s") * 2 + lax.axis_index("c")
    row0 = wid * ROWS_W
    pltpu.sync_copy(lo_hbm, lo_v)
    pltpu.sync_copy(hi_hbm, hi_v)
    lo = lo_v[...]
    span = (hi_v[...] - lo).astype(jnp.uint32)
    lane = lax.iota(jnp.int32, LANES)
    ones = jnp.ones((LANES,), jnp.int32)
    zeros = jnp.zeros((LANES,), jnp.int32)

    @plsc.parallel_loop(0, HWORDS // LANES, unroll=8)
    def _(i):
        hist_v[pl.ds(i * LANES, LANES)] = zeros

    def process(buf):
        @plsc.parallel_loop(0, CHUNK // LANES, unroll=_UNROLL)
        def _(k):
            bits = plsc.bitcast(buf[0, pl.ds(k * LANES, LANES)], jnp.int32)
            if masked:
                d = bits - lo
                idx = lax.shift_right_logical(d, shift) * LANES + lane
                okm = d.astype(jnp.uint32) < span
                plsc.addupdate_scatter(hist_v, [idx], ones, mask=okm)
            else:
                idx = lax.shift_right_logical(bits, shift) * LANES + lane
                plsc.addupdate_scatter(hist_v, [idx], ones)

    def start(c, buf, sem):
        pltpu.make_async_copy(
            acts_hbm.at[pl.ds(row0 + c, 1), :], buf, sem).start()

    def wait(c, buf, sem):
        pltpu.make_async_copy(
            acts_hbm.at[pl.ds(row0 + c, 1), :], buf, sem).wait()

    start(0, buf0, sem0)

    def outer(i, c):
        c0 = i * 2
        start(c0 + 1, buf1, sem1)
        wait(c0, buf0, sem0)
        process(buf0)

        @pl.when(c0 + 2 < NCHUNK)
        def _():
            start(c0 + 2, buf0, sem0)

        wait(c0 + 1, buf1, sem1)
        process(buf1)
        return c

    lax.fori_loop(0, NCHUNK // 2, outer, 0)

    @plsc.parallel_loop(0, NB // LANES, unroll=4)
    def _(g):
        b16 = (g * LANES + lane) * LANES
        acc = plsc.load_gather(hist_v, [b16])
        for j in range(1, LANES):
            acc = acc + plsc.load_gather(hist_v, [b16 + j])
        hred_v[pl.ds(g * LANES, LANES)] = acc

    pltpu.sync_copy(hred_v, hist_hbm.at[wid])


def _make_hist(shift, masked=True):
    mesh = plsc.VectorSubcoreMesh(core_axis_name="c", subcore_axis_name="s")
    return pl.kernel(
        functools.partial(_hist_body, shift, masked),
        out_type=jax.ShapeDtypeStruct((NW, NB), jnp.int32),
        mesh=mesh,
        scratch_types=[
            pltpu.VMEM((LANES,), jnp.int32),
            pltpu.VMEM((LANES,), jnp.int32),
            pltpu.VMEM((1, CHUNK), jnp.float32),
            pltpu.VMEM((1, CHUNK), jnp.float32),
            pltpu.VMEM((HWORDS,), jnp.int32),
            pltpu.VMEM((NB,), jnp.int32),
            pltpu.SemaphoreType.DMA,
            pltpu.SemaphoreType.DMA,
        ],
        compiler_params=pltpu.CompilerParams(needs_layout_passes=False),
    )


_hist_p1 = _make_hist(20, masked=False)
_hist_p2 = _make_hist(8)
_hist_p3 = _make_hist(0)


def _select_bucket(hist, k_req):
    rev = hist[::-1]
    cs = jnp.cumsum(rev)
    pos = jnp.argmax(cs >= k_req)
    bucket = jnp.int32(NB - 1) - pos.astype(jnp.int32)
    above = cs[pos] - rev[pos]
    return bucket, k_req - above


def _merge(h):
    return h.sum(axis=0)


def _find_threshold(acts):
    splat = lambda v: jnp.full((LANES,), v, jnp.int32)
    k_req = jnp.int32(TOTAL_K)
    h1 = _merge(_hist_p1(acts, splat(0), splat(jnp.int32(0x7F800001))))
    b1, k2 = _select_bucket(h1, k_req)
    lo2 = b1 << 20
    hi2 = lo2 + (1 << 20)
    h2 = _merge(_hist_p2(acts, splat(lo2), splat(hi2)))
    b2, k3 = _select_bucket(h2, k2)
    lo3 = lo2 + (b2 << 8)
    hi3 = lo3 + (1 << 8)
    h3 = _merge(_hist_p3(acts, splat(lo3), splat(hi3)))
    b3, _ = _select_bucket(h3, k3)
    t_bits = lo3 + b3
    return lax.bitcast_convert_type(t_bits, jnp.float32)


_DBM = 1024


def _decode_en_body(t_ref, a_ref, w_ref, b_ref, topk_ref, r_ref):
    kstep = pl.program_id(1)
    t = t_ref[0]
    a = a_ref[...]
    masked = jnp.where(a >= t, a, 0.0)
    topk_ref[...] = masked
    p = jnp.dot(masked.astype(jnp.bfloat16), w_ref[...],
                preferred_element_type=jnp.float32)

    @pl.when(kstep == 0)
    def _():
        r_ref[...] = p + b_ref[...]

    @pl.when(kstep != 0)
    def _():
        r_ref[...] += p


def _decode_es_body(t_ref, a_ref, w_ref, b_ref, r_ref):
    kstep = pl.program_id(1)
    t = t_ref[0]
    a = a_ref[...]
    masked = jnp.where(a >= t, a, 0.0)
    p = jnp.dot(masked.astype(jnp.bfloat16), w_ref[...],
                preferred_element_type=jnp.float32)

    @pl.when(kstep == 0)
    def _():
        r_ref[...] = p + b_ref[...]

    @pl.when(kstep != 0)
    def _():
        r_ref[...] += p


def _decode_en(acts, thresh, W_bf16, b_dec, bk):
    grid = (TOKENS // _DBM, DICT // bk)
    return pl.pallas_call(
        _decode_en_body,
        grid=grid,
        in_specs=[
            pl.BlockSpec(memory_space=pltpu.SMEM),
            pl.BlockSpec((_DBM, bk), lambda i, k: (i, k)),
            pl.BlockSpec((bk, ACT), lambda i, k: (k, 0)),
            pl.BlockSpec((1, ACT), lambda i, k: (0, 0)),
        ],
        out_specs=[
            pl.BlockSpec((_DBM, bk), lambda i, k: (i, k)),
            pl.BlockSpec((_DBM, ACT), lambda i, k: (i, 0)),
        ],
        out_shape=[
            jax.ShapeDtypeStruct((TOKENS, DICT), jnp.float32),
            jax.ShapeDtypeStruct((TOKENS, ACT), jnp.float32),
        ],
        compiler_params=pltpu.CompilerParams(
            dimension_semantics=("parallel", "arbitrary"),
            vmem_limit_bytes=63 * 1024 * 1024,
        ),
    )(thresh, acts, W_bf16, b_dec.reshape(1, ACT))


def _decode_es(acts, thresh, W_bf16, b_dec, bk):
    grid = (TOKENS // _DBM, DICT // bk)
    return pl.pallas_call(
        _decode_es_body,
        grid=grid,
        in_specs=[
            pl.BlockSpec(memory_space=pltpu.SMEM),
            pl.BlockSpec((_DBM, bk), lambda i, k: (i, k)),
            pl.BlockSpec((bk, ACT), lambda i, k: (k, 0)),
            pl.BlockSpec((1, ACT), lambda i, k: (0, 0)),
        ],
        out_specs=pl.BlockSpec((_DBM, ACT), lambda i, k: (i, 0)),
        out_shape=jax.ShapeDtypeStruct((TOKENS, ACT), jnp.float32),
        compiler_params=pltpu.CompilerParams(
            dimension_semantics=("parallel", "arbitrary"),
            vmem_limit_bytes=63 * 1024 * 1024,
        ),
    )(thresh, acts, W_bf16, b_dec.reshape(1, ACT))


def kernel(x, W_enc, b_enc, W_dec_en, b_dec_en, W_dec_es, b_dec_es):
    acts = _encode(x, W_enc, b_enc)
    thresh = _find_threshold(acts)
    t1 = thresh.reshape(1)
    acts_topk, recon_en = _decode_en(
        acts, t1, W_dec_en.astype(jnp.bfloat16), b_dec_en, 1024)
    recon_es = _decode_es(
        acts, t1, W_dec_es.astype(jnp.bfloat16), b_dec_es, 2048)
    return recon_en, recon_es, acts_topk


# speedup vs baseline: 1.0059x; 1.0059x over previous
import functools

import jax
import jax.numpy as jnp
from jax import lax
from jax.experimental import pallas as pl
from jax.experimental.pallas import tpu as pltpu
from jax.experimental.pallas import tpu_sc as plsc

ACT = 2048
DICT = 16384
TOKENS = 4096
TOPK_PER_TOKEN = 64
TOTAL = TOKENS * DICT
TOTAL_K = TOKENS * TOPK_PER_TOKEN


_EBM, _EBN = 1024, 1024


def _encode_body(x_ref, w_ref, b_ref, o_ref):
    pre = (jnp.dot(x_ref[...], w_ref[...], preferred_element_type=jnp.float32)
           + b_ref[...])
    o_ref[...] = jnp.where(pre > 0, pre, 0.0)


def _encode(x, W_enc, b_enc):
    grid = (TOKENS // _EBM, DICT // _EBN)
    return pl.pallas_call(
        _encode_body,
        grid=grid,
        in_specs=[
            pl.BlockSpec((_EBM, ACT), lambda i, j: (i, 0)),
            pl.BlockSpec((ACT, _EBN), lambda i, j: (0, j)),
            pl.BlockSpec((1, _EBN), lambda i, j: (0, j)),
        ],
        out_specs=pl.BlockSpec((_EBM, _EBN), lambda i, j: (i, j)),
        out_shape=jax.ShapeDtypeStruct((TOKENS, DICT), jnp.float32),
    )(x, W_enc, b_enc.reshape(1, DICT))


NB = 4096
LANES = 16
HWORDS = NB * LANES
NW = 32
CHUNK = DICT
ROWS_W = TOKENS // NW
NCHUNK = ROWS_W
_UNROLL = 8


def _hist_body(shift, masked, acts_hbm, lo_hbm, hi_hbm, hist_hbm,
               lo_v, hi_v, buf0, buf1, hist_v, sem0, sem1):
    wid = lax.axis_index("s") * 2 + lax.axis_index("c")
    row0 = wid * ROWS_W
    pltpu.sync_copy(lo_hbm, lo_v)
    pltpu.sync_copy(hi_hbm, hi_v)
    lo = lo_v[...]
    span = (hi_v[...] - lo).astype(jnp.uint32)
    lane = lax.iota(jnp.int32, LANES)
    ones = jnp.ones((LANES,), jnp.int32)
    zeros = jnp.zeros((LANES,), jnp.int32)

    @plsc.parallel_loop(0, HWORDS // LANES, unroll=8)
    def _(i):
        hist_v[pl.ds(i * LANES, LANES)] = zeros

    def process(buf):
        @plsc.parallel_loop(0, CHUNK // LANES, unroll=_UNROLL)
        def _(k):
            bits = plsc.bitcast(buf[0, pl.ds(k * LANES, LANES)], jnp.int32)
            if masked:
                d = bits - lo
                idx = lax.shift_right_logical(d, shift) * LANES + lane
                okm = d.astype(jnp.uint32) < span
                plsc.addupdate_scatter(hist_v, [idx], ones, mask=okm)
            else:
                idx = lax.shift_right_logical(bits, shift) * LANES + lane
                plsc.addupdate_scatter(hist_v, [idx], ones)

    def start(c, buf, sem):
        pltpu.make_async_copy(
            acts_hbm.at[pl.ds(row0 + c, 1), :], buf, sem).start()

    def wait(c, buf, sem):
        pltpu.make_async_copy(
            acts_hbm.at[pl.ds(row0 + c, 1), :], buf, sem).wait()

    start(0, buf0, sem0)

    def outer(i, c):
        c0 = i * 2
        start(c0 + 1, buf1, sem1)
        wait(c0, buf0, sem0)
        process(buf0)

        @pl.when(c0 + 2 < NCHUNK)
        def _():
            start(c0 + 2, buf0, sem0)

        wait(c0 + 1, buf1, sem1)
        process(buf1)
        return c

    lax.fori_loop(0, NCHUNK // 2, outer, 0)
    pltpu.sync_copy(hist_v, hist_hbm.at[wid])


def _make_hist(shift, masked=True):
    mesh = plsc.VectorSubcoreMesh(core_axis_name="c", subcore_axis_name="s")
    return pl.kernel(
        functools.partial(_hist_body, shift, masked),
        out_type=jax.ShapeDtypeStruct((NW, HWORDS), jnp.int32),
        mesh=mesh,
        scratch_types=[
            pltpu.VMEM((LANES,), jnp.int32),
            pltpu.VMEM((LANES,), jnp.int32),
            pltpu.VMEM((1, CHUNK), jnp.float32),
            pltpu.VMEM((1, CHUNK), jnp.float32),
            pltpu.VMEM((HWORDS,), jnp.int32),
            pltpu.SemaphoreType.DMA,
            pltpu.SemaphoreType.DMA,
        ],
        compiler_params=pltpu.CompilerParams(needs_layout_passes=False),
    )


_hist_p1 = _make_hist(20, masked=False)
_hist_p2 = _make_hist(8)
_hist_p3 = _make_hist(0)


def _select_bucket(hist, k_req):
    rev = hist[::-1]
    cs = jnp.cumsum(rev)
    pos = jnp.argmax(cs >= k_req)
    bucket = jnp.int32(NB - 1) - pos.astype(jnp.int32)
    above = cs[pos] - rev[pos]
    return bucket, k_req - above


def _merge(h):
    return h.reshape(NW, NB, LANES).sum(axis=(0, 2))


def _find_threshold(acts):
    splat = lambda v: jnp.full((LANES,), v, jnp.int32)
    k_req = jnp.int32(TOTAL_K)
    h1 = _merge(_hist_p1(acts, splat(0), splat(jnp.int32(0x7F800001))))
    b1, k2 = _select_bucket(h1, k_req)
    lo2 = b1 << 20
    hi2 = lo2 + (1 << 20)
    h2 = _merge(_hist_p2(acts, splat(lo2), splat(hi2)))
    b2, k3 = _select_bucket(h2, k2)
    lo3 = lo2 + (b2 << 8)
    hi3 = lo3 + (1 << 8)
    h3 = _merge(_hist_p3(acts, splat(lo3), splat(hi3)))
    b3, _ = _select_bucket(h3, k3)
    t_bits = lo3 + b3
    return lax.bitcast_convert_type(t_bits, jnp.float32)


_DBM = 1024


def _decode_en_body(t_ref, a_ref, w_ref, b_ref, topk_ref, r_ref):
    kstep = pl.program_id(1)
    t = t_ref[0]
    a = a_ref[...]
    masked = jnp.where(a >= t, a, 0.0)
    topk_ref[...] = masked
    p = jnp.dot(masked.astype(jnp.bfloat16), w_ref[...],
                preferred_element_type=jnp.float32)

    @pl.when(kstep == 0)
    def _():
        r_ref[...] = p + b_ref[...]

    @pl.when(kstep != 0)
    def _():
        r_ref[...] += p


def _decode_es_body(t_ref, a_ref, w_ref, b_ref, r_ref):
    kstep = pl.program_id(1)
    t = t_ref[0]
    a = a_ref[...]
    masked = jnp.where(a >= t, a, 0.0)
    p = jnp.dot(masked.astype(jnp.bfloat16), w_ref[...],
                preferred_element_type=jnp.float32)

    @pl.when(kstep == 0)
    def _():
        r_ref[...] = p + b_ref[...]

    @pl.when(kstep != 0)
    def _():
        r_ref[...] += p


def _decode_en(acts, thresh, W_bf16, b_dec, bk):
    grid = (TOKENS // _DBM, DICT // bk)
    return pl.pallas_call(
        _decode_en_body,
        grid=grid,
        in_specs=[
            pl.BlockSpec(memory_space=pltpu.SMEM),
            pl.BlockSpec((_DBM, bk), lambda i, k: (i, k)),
            pl.BlockSpec((bk, ACT), lambda i, k: (k, 0)),
            pl.BlockSpec((1, ACT), lambda i, k: (0, 0)),
        ],
        out_specs=[
            pl.BlockSpec((_DBM, bk), lambda i, k: (i, k)),
            pl.BlockSpec((_DBM, ACT), lambda i, k: (i, 0)),
        ],
        out_shape=[
            jax.ShapeDtypeStruct((TOKENS, DICT), jnp.float32),
            jax.ShapeDtypeStruct((TOKENS, ACT), jnp.float32),
        ],
        compiler_params=pltpu.CompilerParams(
            dimension_semantics=("parallel", "arbitrary"),
            vmem_limit_bytes=63 * 1024 * 1024,
        ),
    )(thresh, acts, W_bf16, b_dec.reshape(1, ACT))


def _decode_es(acts, thresh, W_bf16, b_dec, bk):
    grid = (TOKENS // _DBM, DICT // bk)
    return pl.pallas_call(
        _decode_es_body,
        grid=grid,
        in_specs=[
            pl.BlockSpec(memory_space=pltpu.SMEM),
            pl.BlockSpec((_DBM, bk), lambda i, k: (i, k)),
            pl.BlockSpec((bk, ACT), lambda i, k: (k, 0)),
            pl.BlockSpec((1, ACT), lambda i, k: (0, 0)),
        ],
        out_specs=pl.BlockSpec((_DBM, ACT), lambda i, k: (i, 0)),
        out_shape=jax.ShapeDtypeStruct((TOKENS, ACT), jnp.float32),
        compiler_params=pltpu.CompilerParams(
            dimension_semantics=("parallel", "arbitrary"),
            vmem_limit_bytes=63 * 1024 * 1024,
        ),
    )(thresh, acts, W_bf16, b_dec.reshape(1, ACT))


def kernel(x, W_enc, b_enc, W_dec_en, b_dec_en, W_dec_es, b_dec_es):
    acts = _encode(x, W_enc, b_enc)
    thresh = _find_threshold(acts)
    t1 = thresh.reshape(1)
    acts_topk, recon_en = _decode_en(
        acts, t1, W_dec_en.astype(jnp.bfloat16), b_dec_en, 1024)
    recon_es = _decode_es(
        acts, t1, W_dec_es.astype(jnp.bfloat16), b_dec_es, 2048)
    return recon_en, recon_es, acts_topk
